# Initial kernel scaffold; baseline (speedup 1.0000x reference)
import functools

import jax
import jax.numpy as jnp
from jax import lax
from jax.experimental import pallas as pl
from jax.experimental.pallas import tpu as pltpu
from jax.experimental.pallas import tpu_sc as plsc

N = 10000
E = 320000
D = 128

NC = 2
NS = 16
L = 16
NW = NC * NS

C = 128
CPW = 80
EP = NW * CPW * C
TOTCH = EP // C
NP = 10240
NB = 4
RZ = 125

_MESH = plsc.VectorSubcoreMesh(core_axis_name="c", subcore_axis_name="s")

_SELU_L = 1.0507009873554805
_SELU_A = 1.6732632423543772


def _selu(r):
    return _SELU_L * jnp.where(r > 0, r, _SELU_A * jnp.expm1(r))


@functools.partial(
    pl.kernel,
    out_type=jax.ShapeDtypeStruct((4, NW, NP), jnp.float32),
    mesh=_MESH,
    scratch_types=[
        pltpu.VMEM((CPW, 3, C), jnp.int32),
        pltpu.VMEM((4, NP), jnp.float32),
    ],
)
def _deg_kernel(ed_hbm, out_hbm, ed_v, tabs):
    cid = lax.axis_index("c")
    sid = lax.axis_index("s")
    wid = sid * NC + cid

    @pl.loop(0, NP // L)
    def _zero(i):
        z = jnp.zeros((L,), jnp.float32)
        for t in range(4):
            tabs[t, pl.ds(i * L, L)] = z

    pltpu.sync_copy(ed_hbm.at[pl.ds(wid * CPW, CPW)], ed_v)

    @pl.loop(0, CPW)
    def _chunk(ci):
        ones = jnp.ones((L,), jnp.float32)
        zeros = jnp.zeros((L,), jnp.float32)
        for j in range(C // L):
            sl = pl.ds(j * L, L)
            s16 = ed_v[ci, 0, sl]
            d16 = ed_v[ci, 1, sl]
            wf = plsc.bitcast(ed_v[ci, 2, sl], jnp.float32)
            cnt = jnp.where(wf > 0, ones, zeros)
            plsc.addupdate_scatter(tabs.at[0], [s16], wf)
            plsc.addupdate_scatter(tabs.at[1], [d16], wf)
            plsc.addupdate_scatter(tabs.at[2], [s16], cnt)
            plsc.addupdate_scatter(tabs.at[3], [d16], cnt)

    for t in range(4):
        pltpu.sync_copy(tabs.at[t], out_hbm.at[t, wid])


@functools.partial(
    pl.kernel,
    out_type=jax.ShapeDtypeStruct((NC, N, D), jnp.float32),
    mesh=_MESH,
    scratch_types=[
        pltpu.VMEM_SHARED((N, D), jnp.float32),
        pltpu.VMEM((NB, 3, C), jnp.int32),
        pltpu.VMEM((NB, C, D), jnp.float32),
        pltpu.VMEM((RZ, D), jnp.float32),
        [pltpu.SemaphoreType.DMA] * NB,
        [pltpu.SemaphoreType.DMA] * NB,
    ],
)
def _spmm_kernel(g_hbm, ed_hbm, out_hbm, agg, ed_v, rows_v, zbuf, gsems, ssems):
    cid = lax.axis_index("c")
    sid = lax.axis_index("s")
    wid = sid * NC + cid
    chbase = wid * CPW

    @pl.loop(0, RZ)
    def _zz(i):
        z = jnp.zeros((L,), jnp.float32)
        for j in range(D // L):
            zbuf[i, pl.ds(j * L, L)] = z

    @pl.loop(0, 5)
    def _za(k):
        r0 = (sid * 5 + k) * RZ
        pltpu.sync_copy(zbuf, agg.at[pl.ds(r0, RZ)])

    plsc.subcore_barrier()

    def load_chunk(ci, b):
        pltpu.sync_copy(ed_hbm.at[chbase + ci], ed_v.at[b])

    def start_gather(b):
        pltpu.async_copy(g_hbm.at[ed_v.at[b, 0]], rows_v.at[b], gsems[b])

    def wait_gather(b):
        pltpu.make_async_copy(g_hbm.at[ed_v.at[b, 0]], rows_v.at[b], gsems[b]).wait()

    def start_scatter(b):
        pltpu.async_copy(rows_v.at[b], agg.at[ed_v.at[b, 1]], ssems[b], add=True)

    def wait_scatter(b):
        pltpu.make_async_copy(rows_v.at[b], agg.at[ed_v.at[b, 1]], ssems[b]).wait()

    for b in range(NB - 1):
        load_chunk(b, b)
        start_gather(b)

    @pl.loop(0, CPW, step=NB)
    def _main(i):
        for b in range(NB):
            ci = i + b
            wait_gather(b)

            @pl.loop(0, C)
            def _scale(e):
                wi = plsc.load_gather(
                    ed_v,
                    [jnp.full((L,), b, jnp.int32),
                     jnp.full((L,), 2, jnp.int32),
                     jnp.full((L,), e, jnp.int32)],
                )
                wf = plsc.bitcast(wi, jnp.float32)
                for j in range(D // L):
                    sl = pl.ds(j * L, L)
                    rows_v[b, e, sl] = rows_v[b, e, sl] * wf

            start_scatter(b)
            nb = (b + NB - 1) % NB
            nxt = ci + (NB - 1)

            @pl.when(jnp.logical_and(nxt < CPW, ci >= 1))
            def _w():
                wait_scatter(nb)

            @pl.when(nxt < CPW)
            def _l():
                load_chunk(nxt, nb)
                start_gather(nb)

    for b in range(NB):
        wait_scatter(b)

    plsc.subcore_barrier()

    @pl.loop(0, 5)
    def _dump(k):
        r0 = (sid * 5 + k) * RZ
        pltpu.sync_copy(agg.at[pl.ds(r0, RZ)], out_hbm.at[cid, pl.ds(r0, RZ)])


def _prep_body(ow_r, iw_r, od_r, id_r, s1_r, post_r):
    ow = jnp.sum(ow_r[...], axis=0, keepdims=True)
    iw = jnp.sum(iw_r[...], axis=0, keepdims=True)
    od = jnp.sum(od_r[...], axis=0, keepdims=True)
    idg = jnp.sum(id_r[...], axis=0, keepdims=True)
    s1_r[...] = lax.rsqrt(jnp.maximum(ow, 1e-12)) * lax.rsqrt(jnp.maximum(od, 1.0))
    post_r[...] = lax.rsqrt(jnp.maximum(iw, 1e-12)) * lax.rsqrt(jnp.maximum(idg, 1.0))


_prep_call = pl.pallas_call(
    _prep_body,
    out_shape=(
        jax.ShapeDtypeStruct((1, NP), jnp.float32),
        jax.ShapeDtypeStruct((1, NP), jnp.float32),
    ),
)

_BM = 1000


def _mm1_body(x_r, s_r, w_r, o_r):
    o_r[...] = jnp.dot(x_r[...] * s_r[...], w_r[...],
                       preferred_element_type=jnp.float32)


_mm1_call = pl.pallas_call(
    _mm1_body,
    grid=(N // _BM,),
    in_specs=[
        pl.BlockSpec((_BM, D), lambda i: (i, 0)),
        pl.BlockSpec((_BM, 1), lambda i: (i, 0)),
        pl.BlockSpec((D, D), lambda i: (0, 0)),
    ],
    out_specs=pl.BlockSpec((_BM, D), lambda i: (i, 0)),
    out_shape=jax.ShapeDtypeStruct((N, D), jnp.float32),
)


def _mm2_body(p0_r, p1_r, post_r, b_r, s_r, w_r, o_r):
    rst = (p0_r[...] + p1_r[...]) * post_r[...] + b_r[...]
    o_r[...] = jnp.dot(_selu(rst) * s_r[...], w_r[...],
                       preferred_element_type=jnp.float32)


_mm2_call = pl.pallas_call(
    _mm2_body,
    grid=(N // _BM,),
    in_specs=[
        pl.BlockSpec((_BM, D), lambda i: (i, 0)),
        pl.BlockSpec((_BM, D), lambda i: (i, 0)),
        pl.BlockSpec((_BM, 1), lambda i: (i, 0)),
        pl.BlockSpec((1, D), lambda i: (0, 0)),
        pl.BlockSpec((_BM, 1), lambda i: (i, 0)),
        pl.BlockSpec((D, D), lambda i: (0, 0)),
    ],
    out_specs=pl.BlockSpec((_BM, D), lambda i: (i, 0)),
    out_shape=jax.ShapeDtypeStruct((N, D), jnp.float32),
)


def _final_body(p0_r, p1_r, post_r, b_r, o_r):
    o_r[...] = _selu((p0_r[...] + p1_r[...]) * post_r[...] + b_r[...])


_final_call = pl.pallas_call(
    _final_body,
    grid=(N // _BM,),
    in_specs=[
        pl.BlockSpec((_BM, D), lambda i: (i, 0)),
        pl.BlockSpec((_BM, D), lambda i: (i, 0)),
        pl.BlockSpec((_BM, 1), lambda i: (i, 0)),
        pl.BlockSpec((1, D), lambda i: (0, 0)),
    ],
    out_specs=pl.BlockSpec((_BM, D), lambda i: (i, 0)),
    out_shape=jax.ShapeDtypeStruct((N, D), jnp.float32),
)


def kernel(x, edge_index, edge_weight, W1, b1, W2, b2):
    src = edge_index[0]
    dst = edge_index[1]
    pad = EP - E
    srcp = jnp.concatenate([src, jnp.zeros((pad,), jnp.int32)])
    dstp = jnp.concatenate([dst, jnp.zeros((pad,), jnp.int32)])
    wp = jnp.concatenate([edge_weight, jnp.zeros((pad,), jnp.float32)])
    wbits = lax.bitcast_convert_type(wp, jnp.int32)
    ed = jnp.stack(
        [srcp.reshape(TOTCH, C), dstp.reshape(TOTCH, C), wbits.reshape(TOTCH, C)],
        axis=1,
    )

    parts = _deg_kernel(ed)
    s1m, postm = _prep_call(parts[0], parts[1], parts[2], parts[3])
    s1c = s1m[0, :N][:, None]
    postc = postm[0, :N][:, None]

    g1 = _mm1_call(x, s1c, W1)
    p1 = _spmm_kernel(g1, ed)
    g2 = _mm2_call(p1[0], p1[1], postc, b1[None, :], s1c, W2)
    p2 = _spmm_kernel(g2, ed)
    return _final_call(p2[0], p2[1], postc, b2[None, :])


# speedup vs baseline: 8.2034x; 8.2034x over previous
import functools

import jax
import jax.numpy as jnp
from jax import lax
from jax.experimental import pallas as pl
from jax.experimental.pallas import tpu as pltpu
from jax.experimental.pallas import tpu_sc as plsc

N = 10000
E = 320000
D = 128

NC = 2
NS = 16
L = 16
NW = NC * NS

C = 64
CPW = 160
EP = NW * CPW * C
TOTCH = EP // C
NP = 10240
NB = 4
RPT = 624
RZ = 104

_MESH = plsc.VectorSubcoreMesh(core_axis_name="c", subcore_axis_name="s")

_SELU_L = 1.0507009873554805
_SELU_A = 1.6732632423543772


def _selu(r):
    return _SELU_L * jnp.where(r > 0, r, _SELU_A * (jnp.exp(r) - 1.0))


@functools.partial(
    pl.kernel,
    out_type=jax.ShapeDtypeStruct((4 * NW * NP,), jnp.float32),
    mesh=_MESH,
    scratch_types=[
        pltpu.VMEM((CPW, 2, C), jnp.int32),
        pltpu.VMEM((CPW * C,), jnp.float32),
        [pltpu.VMEM((NP,), jnp.float32)] * 4,
    ],
    compiler_params=pltpu.CompilerParams(needs_layout_passes=False),
)
def _deg_kernel(ed_hbm, w_hbm, out_hbm, ed_v, w_v, tabs):
    cid = lax.axis_index("c")
    sid = lax.axis_index("s")
    wid = sid * NC + cid

    @pl.loop(0, NP // L)
    def _zero(i):
        z = jnp.zeros((L,), jnp.float32)
        for t in range(4):
            tabs[t][pl.ds(i * L, L)] = z

    pltpu.sync_copy(ed_hbm.at[pl.ds(wid * CPW, CPW)], ed_v)
    pltpu.sync_copy(w_hbm.at[pl.ds(wid * CPW * C, CPW * C)], w_v)

    @pl.loop(0, CPW)
    def _chunk(ci):
        ones = jnp.ones((L,), jnp.float32)
        zeros = jnp.zeros((L,), jnp.float32)
        for j in range(C // L):
            sl = pl.ds(j * L, L)
            s16 = ed_v[ci, 0, sl]
            d16 = ed_v[ci, 1, sl]
            wf = w_v[pl.ds(ci * C + j * L, L)]
            cnt = jnp.where(wf > 0, ones, zeros)
            plsc.addupdate_scatter(tabs[0], [s16], wf)
            plsc.addupdate_scatter(tabs[1], [d16], wf)
            plsc.addupdate_scatter(tabs[2], [s16], cnt)
            plsc.addupdate_scatter(tabs[3], [d16], cnt)

    for t in range(4):
        pltpu.sync_copy(tabs[t], out_hbm.at[pl.ds((t * NW + wid) * NP, NP)])


@functools.partial(
    pl.kernel,
    out_type=jax.ShapeDtypeStruct((NC, N, D), jnp.float32),
    mesh=_MESH,
    scratch_types=[
        pltpu.VMEM_SHARED((N, D), jnp.float32),
        pltpu.VMEM((NB, 2, C), jnp.int32),
        pltpu.VMEM((NB, C), jnp.float32),
        pltpu.VMEM((NB, C, D), jnp.float32),
        pltpu.VMEM((RZ, D), jnp.float32),
        [pltpu.SemaphoreType.DMA] * NB,
        [pltpu.SemaphoreType.DMA] * NB,
    ],
    compiler_params=pltpu.CompilerParams(needs_layout_passes=False),
)
def _spmm_kernel(g_hbm, ed_hbm, w_hbm, out_hbm, agg, ed_v, w_v, rows_v, zbuf,
                 gsems, ssems):
    cid = lax.axis_index("c")
    sid = lax.axis_index("s")
    wid = sid * NC + cid
    chbase = wid * CPW

    @pl.loop(0, RZ)
    def _zz(i):
        z = jnp.zeros((L,), jnp.float32)
        for j in range(D // L):
            zbuf[i, pl.ds(j * L, L)] = z

    for k in range(RPT // RZ):
        pltpu.sync_copy(zbuf, agg.at[pl.ds(sid * RPT + k * RZ, RZ)])

    @pl.when(sid == 0)
    def _ztail():
        pltpu.sync_copy(zbuf.at[pl.ds(0, N - NS * RPT)],
                        agg.at[pl.ds(NS * RPT, N - NS * RPT)])

    plsc.subcore_barrier()

    def load_chunk(ci, b):
        pltpu.sync_copy(ed_hbm.at[chbase + ci], ed_v.at[b])
        pltpu.sync_copy(w_hbm.at[pl.ds((chbase + ci) * C, C)], w_v.at[b])

    def start_gather(b):
        pltpu.async_copy(g_hbm.at[ed_v.at[b, 0]], rows_v.at[b], gsems[b])

    def wait_gather(b):
        pltpu.make_async_copy(g_hbm.at[ed_v.at[b, 0]], rows_v.at[b], gsems[b]).wait()

    def start_scatter(b):
        pltpu.async_copy(rows_v.at[b], agg.at[ed_v.at[b, 1]], ssems[b], add=True)

    def wait_scatter(b):
        pltpu.make_async_copy(rows_v.at[b], agg.at[ed_v.at[b, 1]], ssems[b]).wait()

    for b in range(NB - 1):
        load_chunk(b, b)
        start_gather(b)

    @pl.loop(0, CPW, step=NB)
    def _main(i):
        for b in range(NB):
            ci = i + b
            wait_gather(b)

            @pl.loop(0, C)
            def _scale(e):
                wf = plsc.load_gather(
                    w_v,
                    [jnp.full((L,), b, jnp.int32),
                     jnp.full((L,), e, jnp.int32)],
                )
                for j in range(D // L):
                    sl = pl.ds(j * L, L)
                    rows_v[b, e, sl] = rows_v[b, e, sl] * wf

            start_scatter(b)
            nb = (b + NB - 1) % NB
            nxt = ci + (NB - 1)

            @pl.when(jnp.logical_and(nxt < CPW, ci >= 1))
            def _w():
                wait_scatter(nb)

            @pl.when(nxt < CPW)
            def _l():
                load_chunk(nxt, nb)
                start_gather(nb)

    for b in range(NB):
        wait_scatter(b)

    plsc.subcore_barrier()

    pltpu.sync_copy(agg.at[pl.ds(sid * RPT, RPT)],
                    out_hbm.at[cid, pl.ds(sid * RPT, RPT)])

    @pl.when(sid == 0)
    def _dtail():
        pltpu.sync_copy(agg.at[pl.ds(NS * RPT, N - NS * RPT)],
                        out_hbm.at[cid, pl.ds(NS * RPT, N - NS * RPT)])


def _prep_body(ow_r, iw_r, od_r, id_r, s1_r, post_r):
    ow = jnp.sum(ow_r[...], axis=0, keepdims=True)
    iw = jnp.sum(iw_r[...], axis=0, keepdims=True)
    od = jnp.sum(od_r[...], axis=0, keepdims=True)
    idg = jnp.sum(id_r[...], axis=0, keepdims=True)
    s1_r[...] = lax.rsqrt(jnp.maximum(ow, 1e-12)) * lax.rsqrt(jnp.maximum(od, 1.0))
    post_r[...] = lax.rsqrt(jnp.maximum(iw, 1e-12)) * lax.rsqrt(jnp.maximum(idg, 1.0))


_prep_call = pl.pallas_call(
    _prep_body,
    out_shape=(
        jax.ShapeDtypeStruct((1, NP), jnp.float32),
        jax.ShapeDtypeStruct((1, NP), jnp.float32),
    ),
)

_BM = 1000


def _mm1_body(x_r, s_r, w_r, o_r):
    o_r[...] = jnp.dot(x_r[...] * s_r[...], w_r[...],
                       preferred_element_type=jnp.float32)


_mm1_call = pl.pallas_call(
    _mm1_body,
    grid=(N // _BM,),
    in_specs=[
        pl.BlockSpec((_BM, D), lambda i: (i, 0)),
        pl.BlockSpec((_BM, 1), lambda i: (i, 0)),
        pl.BlockSpec((D, D), lambda i: (0, 0)),
    ],
    out_specs=pl.BlockSpec((_BM, D), lambda i: (i, 0)),
    out_shape=jax.ShapeDtypeStruct((N, D), jnp.float32),
)


def _mm2_body(p0_r, p1_r, post_r, b_r, s_r, w_r, o_r):
    rst = (p0_r[...] + p1_r[...]) * post_r[...] + b_r[...]
    o_r[...] = jnp.dot(_selu(rst) * s_r[...], w_r[...],
                       preferred_element_type=jnp.float32)


_mm2_call = pl.pallas_call(
    _mm2_body,
    grid=(N // _BM,),
    in_specs=[
        pl.BlockSpec((_BM, D), lambda i: (i, 0)),
        pl.BlockSpec((_BM, D), lambda i: (i, 0)),
        pl.BlockSpec((_BM, 1), lambda i: (i, 0)),
        pl.BlockSpec((1, D), lambda i: (0, 0)),
        pl.BlockSpec((_BM, 1), lambda i: (i, 0)),
        pl.BlockSpec((D, D), lambda i: (0, 0)),
    ],
    out_specs=pl.BlockSpec((_BM, D), lambda i: (i, 0)),
    out_shape=jax.ShapeDtypeStruct((N, D), jnp.float32),
)


def _final_body(p0_r, p1_r, post_r, b_r, o_r):
    o_r[...] = _selu((p0_r[...] + p1_r[...]) * post_r[...] + b_r[...])


_final_call = pl.pallas_call(
    _final_body,
    grid=(N // _BM,),
    in_specs=[
        pl.BlockSpec((_BM, D), lambda i: (i, 0)),
        pl.BlockSpec((_BM, D), lambda i: (i, 0)),
        pl.BlockSpec((_BM, 1), lambda i: (i, 0)),
        pl.BlockSpec((1, D), lambda i: (0, 0)),
    ],
    out_specs=pl.BlockSpec((_BM, D), lambda i: (i, 0)),
    out_shape=jax.ShapeDtypeStruct((N, D), jnp.float32),
)


def kernel(x, edge_index, edge_weight, W1, b1, W2, b2):
    src = edge_index[0]
    dst = edge_index[1]
    pad = EP - E
    srcp = jnp.concatenate([src, jnp.zeros((pad,), jnp.int32)])
    dstp = jnp.concatenate([dst, jnp.zeros((pad,), jnp.int32)])
    wp = jnp.concatenate([edge_weight, jnp.zeros((pad,), jnp.float32)])
    ed = jnp.stack([srcp.reshape(TOTCH, C), dstp.reshape(TOTCH, C)], axis=1)

    parts = _deg_kernel(ed, wp).reshape(4, NW, NP)
    s1m, postm = _prep_call(parts[0], parts[1], parts[2], parts[3])
    s1c = s1m[0, :N][:, None]
    postc = postm[0, :N][:, None]

    g1 = _mm1_call(x, s1c, W1)
    p1 = _spmm_kernel(g1, ed, wp)
    g2 = _mm2_call(p1[0], p1[1], postc, b1[None, :], s1c, W2)
    p2 = _spmm_kernel(g2, ed, wp)
    return _final_call(p2[0], p2[1], postc, b2[None, :])


# speedup vs baseline: 8.6594x; 1.0556x over previous
import functools

import jax
import jax.numpy as jnp
from jax import lax
from jax.experimental import pallas as pl
from jax.experimental.pallas import tpu as pltpu
from jax.experimental.pallas import tpu_sc as plsc

N = 10000
E = 320000
D = 128

NC = 2
NS = 16
L = 16
NW = NC * NS

C = 64
CPW = 160
EP = NW * CPW * C
TOTCH = EP // C
NP = 10240
NB = 4
RPT = 624
RZ = 96

_MESH = plsc.VectorSubcoreMesh(core_axis_name="c", subcore_axis_name="s")

_SELU_L = 1.0507009873554805
_SELU_A = 1.6732632423543772


def _selu(r):
    return _SELU_L * jnp.where(r > 0, r, _SELU_A * (jnp.exp(r) - 1.0))


@functools.partial(
    pl.kernel,
    out_type=jax.ShapeDtypeStruct((4 * NW * NP,), jnp.float32),
    mesh=_MESH,
    scratch_types=[
        pltpu.VMEM((CPW, 2, C), jnp.int32),
        pltpu.VMEM((CPW * C,), jnp.float32),
        [pltpu.VMEM((NP,), jnp.float32)] * 4,
    ],
    compiler_params=pltpu.CompilerParams(needs_layout_passes=False),
)
def _deg_kernel(ed_hbm, w_hbm, out_hbm, ed_v, w_v, tabs):
    cid = lax.axis_index("c")
    sid = lax.axis_index("s")
    wid = sid * NC + cid

    @pl.loop(0, NP // L)
    def _zero(i):
        z = jnp.zeros((L,), jnp.float32)
        for t in range(4):
            tabs[t][pl.ds(i * L, L)] = z

    pltpu.sync_copy(ed_hbm.at[pl.ds(wid * CPW, CPW)], ed_v)
    pltpu.sync_copy(w_hbm.at[pl.ds(wid * CPW * C, CPW * C)], w_v)

    @pl.loop(0, CPW)
    def _chunk(ci):
        ones = jnp.ones((L,), jnp.float32)
        zeros = jnp.zeros((L,), jnp.float32)
        for j in range(C // L):
            sl = pl.ds(j * L, L)
            s16 = ed_v[ci, 0, sl]
            d16 = ed_v[ci, 1, sl]
            wf = w_v[pl.ds(ci * C + j * L, L)]
            cnt = jnp.where(wf > 0, ones, zeros)
            plsc.addupdate_scatter(tabs[0], [s16], wf)
            plsc.addupdate_scatter(tabs[1], [d16], wf)
            plsc.addupdate_scatter(tabs[2], [s16], cnt)
            plsc.addupdate_scatter(tabs[3], [d16], cnt)

    for t in range(4):
        pltpu.sync_copy(tabs[t], out_hbm.at[pl.ds((t * NW + wid) * NP, NP)])


G = 8


@functools.partial(
    pl.kernel,
    out_type=jax.ShapeDtypeStruct((NC, N, D), jnp.float32),
    mesh=_MESH,
    scratch_types=[
        pltpu.VMEM_SHARED((N, D), jnp.float32),
        pltpu.VMEM((2, G, 2, C), jnp.int32),
        pltpu.VMEM((2, G * C), jnp.float32),
        pltpu.VMEM((NB, C, D), jnp.float32),
        pltpu.VMEM((RZ, D), jnp.float32),
        [pltpu.SemaphoreType.DMA] * NB,
        [pltpu.SemaphoreType.DMA] * NB,
        [pltpu.SemaphoreType.DMA] * 2,
    ],
    compiler_params=pltpu.CompilerParams(needs_layout_passes=False),
)
def _spmm_kernel(g_hbm, ed_hbm, w_hbm, out_hbm, agg, sed, sw, rows_v, zbuf,
                 gsems, ssems, lsems):
    cid = lax.axis_index("c")
    sid = lax.axis_index("s")
    wid = sid * NC + cid
    chbase = wid * CPW

    @pl.loop(0, RZ)
    def _zz(i):
        z = jnp.zeros((L,), jnp.float32)
        for j in range(D // L):
            zbuf[i, pl.ds(j * L, L)] = z

    for k in range(RPT // RZ):
        pltpu.sync_copy(zbuf, agg.at[pl.ds(sid * RPT + k * RZ, RZ)])
    _zrem = RPT - (RPT // RZ) * RZ
    if _zrem:
        pltpu.sync_copy(zbuf.at[pl.ds(0, _zrem)],
                        agg.at[pl.ds(sid * RPT + RPT - _zrem, _zrem)])

    @pl.when(sid == 0)
    def _ztail():
        pltpu.sync_copy(zbuf.at[pl.ds(0, N - NS * RPT)],
                        agg.at[pl.ds(NS * RPT, N - NS * RPT)])

    plsc.subcore_barrier()

    def slab_copies(cg, par):
        return (
            pltpu.make_async_copy(ed_hbm.at[pl.ds(chbase + cg, G)],
                                  sed.at[par], lsems[par]),
            pltpu.make_async_copy(w_hbm.at[pl.ds((chbase + cg) * C, G * C)],
                                  sw.at[par], lsems[par]),
        )

    def start_slab(cg, par):
        a, b = slab_copies(cg, par)
        a.start()
        b.start()

    def wait_slab(cg, par):
        a, b = slab_copies(cg, par)
        a.wait()
        b.wait()

    def start_gather(b, par, row):
        pltpu.async_copy(g_hbm.at[sed.at[par, row, 0]], rows_v.at[b], gsems[b])

    def wait_gather(b, par, row):
        pltpu.make_async_copy(g_hbm.at[sed.at[par, row, 0]], rows_v.at[b],
                              gsems[b]).wait()

    def start_scatter(b, par, row):
        pltpu.async_copy(rows_v.at[b], agg.at[sed.at[par, row, 1]], ssems[b],
                         add=True)

    def wait_scatter(b, par, row):
        pltpu.make_async_copy(rows_v.at[b], agg.at[sed.at[par, row, 1]],
                              ssems[b]).wait()

    start_slab(0, 0)
    start_slab(G, 1)
    wait_slab(0, 0)
    for b in range(NB - 1):
        start_gather(b, 0, b)
    wait_slab(G, 1)

    @pl.loop(0, CPW, step=2 * G)
    def _main(i):
        for par in range(2):
            cb = i + par * G
            for b8 in range(G):
                b = b8 % NB
                ci = cb + b8

                if b8 == 1:
                    @pl.when(cb + G < CPW)
                    def _ls():
                        start_slab(cb + G, 1 - par)

                if b8 == 4:
                    @pl.when(cb + G < CPW)
                    def _lw():
                        wait_slab(cb + G, 1 - par)

                wait_gather(b, par, b8)

                @pl.loop(0, C)
                def _scale(e):
                    wf = plsc.load_gather(
                        sw,
                        [jnp.full((L,), par, jnp.int32),
                         jnp.full((L,), e, jnp.int32) + (b8 * C)],
                    )
                    for j in range(D // L):
                        sl = pl.ds(j * L, L)
                        rows_v[b, e, sl] = rows_v[b, e, sl] * wf

                start_scatter(b, par, b8)

                nb = (b + NB - 1) % NB
                j8 = b8 + NB - 1
                npar, nrow = (par, j8) if j8 < G else (1 - par, j8 - G)
                nxt = ci + NB - 1

                @pl.when(jnp.logical_and(nxt < CPW, ci >= 1))
                def _w():
                    wait_scatter(nb, npar, nrow)

                @pl.when(nxt < CPW)
                def _l():
                    start_gather(nb, npar, nrow)

    for b in range(NB):
        wait_scatter(b, 0, b)

    plsc.subcore_barrier()

    pltpu.sync_copy(agg.at[pl.ds(sid * RPT, RPT)],
                    out_hbm.at[cid, pl.ds(sid * RPT, RPT)])

    @pl.when(sid == 0)
    def _dtail():
        pltpu.sync_copy(agg.at[pl.ds(NS * RPT, N - NS * RPT)],
                        out_hbm.at[cid, pl.ds(NS * RPT, N - NS * RPT)])


def _prep_body(ow_r, iw_r, od_r, id_r, s1_r, post_r):
    ow = jnp.sum(ow_r[...], axis=0, keepdims=True)
    iw = jnp.sum(iw_r[...], axis=0, keepdims=True)
    od = jnp.sum(od_r[...], axis=0, keepdims=True)
    idg = jnp.sum(id_r[...], axis=0, keepdims=True)
    s1_r[...] = lax.rsqrt(jnp.maximum(ow, 1e-12)) * lax.rsqrt(jnp.maximum(od, 1.0))
    post_r[...] = lax.rsqrt(jnp.maximum(iw, 1e-12)) * lax.rsqrt(jnp.maximum(idg, 1.0))


_prep_call = pl.pallas_call(
    _prep_body,
    out_shape=(
        jax.ShapeDtypeStruct((1, NP), jnp.float32),
        jax.ShapeDtypeStruct((1, NP), jnp.float32),
    ),
)

_BM = 1000


def _mm1_body(x_r, s_r, w_r, o_r):
    o_r[...] = jnp.dot(x_r[...] * s_r[...], w_r[...],
                       preferred_element_type=jnp.float32)


_mm1_call = pl.pallas_call(
    _mm1_body,
    grid=(N // _BM,),
    in_specs=[
        pl.BlockSpec((_BM, D), lambda i: (i, 0)),
        pl.BlockSpec((_BM, 1), lambda i: (i, 0)),
        pl.BlockSpec((D, D), lambda i: (0, 0)),
    ],
    out_specs=pl.BlockSpec((_BM, D), lambda i: (i, 0)),
    out_shape=jax.ShapeDtypeStruct((N, D), jnp.float32),
)


def _mm2_body(p0_r, p1_r, post_r, b_r, s_r, w_r, o_r):
    rst = (p0_r[...] + p1_r[...]) * post_r[...] + b_r[...]
    o_r[...] = jnp.dot(_selu(rst) * s_r[...], w_r[...],
                       preferred_element_type=jnp.float32)


_mm2_call = pl.pallas_call(
    _mm2_body,
    grid=(N // _BM,),
    in_specs=[
        pl.BlockSpec((_BM, D), lambda i: (i, 0)),
        pl.BlockSpec((_BM, D), lambda i: (i, 0)),
        pl.BlockSpec((_BM, 1), lambda i: (i, 0)),
        pl.BlockSpec((1, D), lambda i: (0, 0)),
        pl.BlockSpec((_BM, 1), lambda i: (i, 0)),
        pl.BlockSpec((D, D), lambda i: (0, 0)),
    ],
    out_specs=pl.BlockSpec((_BM, D), lambda i: (i, 0)),
    out_shape=jax.ShapeDtypeStruct((N, D), jnp.float32),
)


def _final_body(p0_r, p1_r, post_r, b_r, o_r):
    o_r[...] = _selu((p0_r[...] + p1_r[...]) * post_r[...] + b_r[...])


_final_call = pl.pallas_call(
    _final_body,
    grid=(N // _BM,),
    in_specs=[
        pl.BlockSpec((_BM, D), lambda i: (i, 0)),
        pl.BlockSpec((_BM, D), lambda i: (i, 0)),
        pl.BlockSpec((_BM, 1), lambda i: (i, 0)),
        pl.BlockSpec((1, D), lambda i: (0, 0)),
    ],
    out_specs=pl.BlockSpec((_BM, D), lambda i: (i, 0)),
    out_shape=jax.ShapeDtypeStruct((N, D), jnp.float32),
)


def kernel(x, edge_index, edge_weight, W1, b1, W2, b2):
    src = edge_index[0]
    dst = edge_index[1]
    pad = EP - E
    srcp = jnp.concatenate([src, jnp.zeros((pad,), jnp.int32)])
    dstp = jnp.concatenate([dst, jnp.zeros((pad,), jnp.int32)])
    wp = jnp.concatenate([edge_weight, jnp.zeros((pad,), jnp.float32)])
    ed = jnp.stack([srcp.reshape(TOTCH, C), dstp.reshape(TOTCH, C)], axis=1)

    parts = _deg_kernel(ed, wp).reshape(4, NW, NP)
    s1m, postm = _prep_call(parts[0], parts[1], parts[2], parts[3])
    s1c = s1m[0, :N][:, None]
    postc = postm[0, :N][:, None]

    g1 = _mm1_call(x, s1c, W1)
    p1 = _spmm_kernel(g1, ed, wp)
    g2 = _mm2_call(p1[0], p1[1], postc, b1[None, :], s1c, W2)
    p2 = _spmm_kernel(g2, ed, wp)
    return _final_call(p2[0], p2[1], postc, b2[None, :])
